# baseline (speedup 1.0000x reference)
import functools

import jax
import jax.numpy as jnp
from jax import lax
from jax.experimental import pallas as pl
from jax.experimental.pallas import tpu as pltpu
from jax.experimental.pallas import tpu_sc as plsc

_TAU = 2048


def _build_sc_kernel(B, E, H, L, NC):
    tail = _TAU * H
    out_e = E + tail
    nchunks = tail // L
    shift = 16
    while (1 << shift) < tail * H:
        shift += 1
    mult = -(-(1 << shift) // H)
    for j in (0, 1, H - 1, H, tail - 2, tail - 1):
        assert (j * mult) >> shift == j // H

    mesh = plsc.VectorSubcoreMesh(core_axis_name="c", subcore_axis_name="s")

    @functools.partial(
        pl.kernel,
        mesh=mesh,
        out_type=(
            jax.ShapeDtypeStruct((B, 2, out_e), jnp.int32),
            jax.ShapeDtypeStruct((B, 1, out_e), jnp.float32),
        ),
        scratch_types=[
            pltpu.VMEM((tail,), jnp.int32),
            pltpu.VMEM((tail,), jnp.float32),
            pltpu.VMEM((B + H, L), jnp.int32),
        ],
    )
    def sc_k(e_hbm, w_hbm, params_hbm, eout_hbm, wout_hbm, tl_i, tl_f, par_v):
        c = lax.axis_index("c")
        s = lax.axis_index("s")
        w = s * NC + c
        b = lax.div(w, 3)
        kind = lax.rem(w, 3)

        @pl.when(w < 3 * B)
        def _active():
            pltpu.sync_copy(params_hbm, par_v)

            @pl.when(kind < 2)
            def _edge_row():
                pltpu.sync_copy(e_hbm.at[b, kind], eout_hbm.at[b, kind, pl.ds(0, E)])
                base_v = par_v[b]
                hsp = [par_v[B + h] for h in range(H)]
                lanes = lax.broadcasted_iota(jnp.int32, (L,), 0)
                kind_v = jnp.full((L,), kind, jnp.int32)

                def body(ci, carry):
                    j = ci * L + lanes
                    t = (j * mult) >> shift
                    r = j - t * H
                    hop = hsp[H - 1]
                    for h in range(H - 2, -1, -1):
                        hop = jnp.where(r == h, hsp[h], hop)
                    tl_i[pl.ds(ci * L, L)] = base_v + t - kind_v * hop
                    return carry

                lax.fori_loop(0, nchunks, body, 0)
                pltpu.sync_copy(tl_i, eout_hbm.at[b, kind, pl.ds(E, tail)])

            @pl.when(kind == 2)
            def _weight_row():
                pltpu.sync_copy(w_hbm.at[b, 0], wout_hbm.at[b, 0, pl.ds(0, E)])
                zeros = jnp.zeros((L,), jnp.float32)

                def zbody(ci, carry):
                    tl_f[pl.ds(ci * L, L)] = zeros
                    return carry

                lax.fori_loop(0, nchunks, zbody, 0)
                pltpu.sync_copy(tl_f, wout_hbm.at[b, 0, pl.ds(E, tail)])

    return sc_k


def kernel(nodes, edges, weights, T, taus, hops):
    del nodes
    B, _, E = edges.shape
    H = hops.shape[0]
    edtype = edges.dtype

    info = plsc.get_sparse_core_info()
    NC, L = info.num_cores, info.num_lanes

    base = T.astype(jnp.int32) + taus.astype(jnp.int32) - _TAU
    scal = jnp.concatenate([base, hops.astype(jnp.int32)])
    params = jnp.broadcast_to(scal[:, None], (B + H, L))

    sc_k = _build_sc_kernel(B, E, H, L, NC)
    edges_out, weights_out = sc_k(edges.astype(jnp.int32), weights, params)
    return edges_out.astype(edtype), weights_out


# speedup vs baseline: 7.2224x; 7.2224x over previous
import functools

import jax
import jax.numpy as jnp
import numpy as np
from jax import lax
from jax.experimental import pallas as pl
from jax.experimental.pallas import tpu as pltpu
from jax.experimental.pallas import tpu_sc as plsc

_TAU = 2048


def _build_sc_kernel(B, E, H, L, NC, NS):
    NW = NC * NS
    tail = _TAU * H
    out_e = E + tail
    R = 3 * B
    NCH = 4 * R
    CPW = NCH // NW
    C = E // 4
    period = H * L
    nper = tail // period
    assert tail % period == 0 and NCH % NW == 0 and E % 4 == 0

    shift = 16
    mult = -(-(1 << shift) // H)
    for j in range(period):
        assert (j * mult) >> shift == j // H

    mesh = plsc.VectorSubcoreMesh(core_axis_name="c", subcore_axis_name="s")

    @functools.partial(
        pl.kernel,
        mesh=mesh,
        out_type=(
            jax.ShapeDtypeStruct((B, 2, out_e), jnp.int32),
            jax.ShapeDtypeStruct((B, 1, out_e), jnp.int32),
        ),
        scratch_types=[
            pltpu.VMEM((CPW * C,), jnp.int32),
            pltpu.VMEM((tail,), jnp.int32),
            pltpu.VMEM((B + H, L), jnp.int32),
            pltpu.SemaphoreType.DMA,
            pltpu.SemaphoreType.DMA,
        ],
    )
    def sc_k(e_hbm, w_hbm, params_hbm, eout_hbm, wout_hbm, buf, tl, par_v, sem_i, sem_o):
        c = lax.axis_index("c")
        s = lax.axis_index("s")
        w = s * NC + c
        b = lax.div(w, 3)
        kind = lax.rem(w, 3)
        is_edge_tail = jnp.logical_and(w < R, kind < 2)
        is_wt_tail = jnp.logical_and(w < R, kind == 2)

        for qi in range(CPW):
            q = w + NW * qi
            row = lax.div(q, 4)
            part = lax.rem(q, 4)
            qb = lax.div(row, 3)
            qk = lax.rem(row, 3)

            @pl.when(qk < 2)
            def _(qi=qi, qb=qb, qk=qk, part=part):
                pltpu.async_copy(
                    e_hbm.at[qb, qk, pl.ds(part * C, C)], buf.at[pl.ds(qi * C, C)], sem_i
                )

            @pl.when(qk == 2)
            def _(qi=qi, qb=qb, part=part):
                pltpu.async_copy(
                    w_hbm.at[qb, 0, pl.ds(part * C, C)], buf.at[pl.ds(qi * C, C)], sem_i
                )

        @pl.when(is_edge_tail)
        def _edge_tail():
            pltpu.sync_copy(params_hbm, par_v)
            base_v = par_v[b]
            kind_v = jnp.full((L,), kind, jnp.int32)
            lanes = lax.broadcasted_iota(jnp.int32, (L,), 0)
            seeds = []
            for h in range(H):
                j = h * L + lanes
                t = (j * mult) >> shift
                r = j - t * H
                hop = par_v[B + H - 1]
                for hh in range(H - 2, -1, -1):
                    hop = jnp.where(r == hh, par_v[B + hh], hop)
                seeds.append(base_v + t - kind_v * hop)

            def body(ci, carry):
                off = ci * period
                vs = carry
                for h in range(H):
                    tl[pl.ds(off + h * L, L)] = vs[h]
                return tuple(v + L for v in vs)

            lax.fori_loop(0, nper, body, tuple(seeds))

        @pl.when(is_wt_tail)
        def _weight_tail():
            zero = jnp.zeros((L,), jnp.int32)

            def zbody(ci, carry):
                off = ci * period
                for h in range(H):
                    tl[pl.ds(off + h * L, L)] = zero
                return carry

            lax.fori_loop(0, nper, zbody, 0)

        for qi in range(CPW):
            pltpu.make_async_copy(
                e_hbm.at[0, 0, pl.ds(0, C)], buf.at[pl.ds(qi * C, C)], sem_i
            ).wait()

        for qi in range(CPW):
            q = w + NW * qi
            row = lax.div(q, 4)
            part = lax.rem(q, 4)
            qb = lax.div(row, 3)
            qk = lax.rem(row, 3)

            @pl.when(qk < 2)
            def _(qi=qi, qb=qb, qk=qk, part=part):
                pltpu.async_copy(
                    buf.at[pl.ds(qi * C, C)], eout_hbm.at[qb, qk, pl.ds(part * C, C)], sem_o
                )

            @pl.when(qk == 2)
            def _(qi=qi, qb=qb, part=part):
                pltpu.async_copy(
                    buf.at[pl.ds(qi * C, C)], wout_hbm.at[qb, 0, pl.ds(part * C, C)], sem_o
                )

        @pl.when(is_edge_tail)
        def _edge_tail_out():
            pltpu.sync_copy(tl, eout_hbm.at[b, kind, pl.ds(E, tail)])

        @pl.when(is_wt_tail)
        def _weight_tail_out():
            pltpu.sync_copy(tl, wout_hbm.at[b, 0, pl.ds(E, tail)])

        for qi in range(CPW):
            pltpu.make_async_copy(
                buf.at[pl.ds(qi * C, C)], eout_hbm.at[0, 0, pl.ds(0, C)], sem_o
            ).wait()

    return sc_k


def kernel(nodes, edges, weights, T, taus, hops):
    del nodes
    B, _, E = edges.shape
    H = hops.shape[0]
    edtype = edges.dtype

    info = plsc.get_sparse_core_info()
    NC, NS, L = info.num_cores, info.num_subcores, info.num_lanes

    base = T.astype(jnp.int32) + taus.astype(jnp.int32) - _TAU
    scal = jnp.concatenate([base, hops.astype(jnp.int32)])
    params = jnp.broadcast_to(scal[:, None], (B + H, L))

    sc_k = _build_sc_kernel(B, E, H, L, NC, NS)
    edges_out, weights_bits = sc_k(
        edges.astype(jnp.int32),
        lax.bitcast_convert_type(weights, jnp.int32),
        params,
    )
    weights_out = lax.bitcast_convert_type(weights_bits, weights.dtype)
    return edges_out.astype(edtype), weights_out
